# Initial kernel scaffold; baseline (speedup 1.0000x reference)
import functools

import jax
import jax.numpy as jnp
from jax import lax
from jax.experimental import pallas as pl
from jax.experimental.pallas import tpu as pltpu
from jax.experimental.pallas import tpu_sc as plsc

NC = 2
NS = 16
NW = NC * NS
LANES = 16


@functools.partial(jax.jit, static_argnames=("batch", "hist_pad", "dim"))
def _mean_emb(x_flat, table, *, batch, hist_pad, dim):
    rw = batch // NW
    r_chunk = 4
    ng = rw // r_chunk
    ch = 104
    nch = r_chunk * hist_pad // ch
    kmax = hist_pad // LANES

    mesh = plsc.VectorSubcoreMesh(core_axis_name="c", subcore_axis_name="s")

    @functools.partial(
        pl.kernel,
        out_type=jax.ShapeDtypeStruct((batch, dim), jnp.float32),
        mesh=mesh,
        scratch_types=[
            pltpu.VMEM((rw * hist_pad,), jnp.int32),
            pltpu.VMEM((r_chunk * hist_pad, dim), jnp.float32),
            pltpu.VMEM((r_chunk * hist_pad, dim), jnp.float32),
            pltpu.VMEM((rw, dim), jnp.float32),
            pltpu.SemaphoreType.DMA,
            pltpu.SemaphoreType.DMA,
        ],
    )
    def k(x_hbm, table_hbm, out_hbm, idx_v, rows0, rows1, out_v, sem0, sem1):
        wid = lax.axis_index("s") * NC + lax.axis_index("c")
        ibase = wid * (rw * hist_pad)
        pltpu.sync_copy(x_hbm.at[pl.ds(ibase, rw * hist_pad)], idx_v)

        rows_bufs = (rows0, rows1)
        sems = (sem0, sem1)

        def fire(g, b):
            goff = g * (r_chunk * hist_pad)
            for j in range(nch):
                pltpu.async_copy(
                    table_hbm.at[idx_v.at[pl.ds(goff + j * ch, ch)]],
                    rows_bufs[b].at[pl.ds(j * ch, ch)],
                    sems[b],
                )

        def drain(b):
            for j in range(nch):
                pltpu.make_async_copy(
                    table_hbm.at[idx_v.at[pl.ds(j * ch, ch)]],
                    rows_bufs[b].at[pl.ds(j * ch, ch)],
                    sems[b],
                ).wait()

        def reduce(g, b):
            rows_v = rows_bufs[b]
            goff = g * (r_chunk * hist_pad)
            for r in range(r_chunk):
                rbase = r * hist_pad

                def kbody(kk, carry, rbase=rbase):
                    a0, a1, cnt = carry
                    o = rbase + kk * LANES
                    idx = idx_v[pl.ds(goff + o, LANES)]
                    cnt = cnt + jnp.where(idx != 0, 1, 0).astype(jnp.int32)
                    for t in range(LANES):
                        a0 = a0 + rows_v[o + t, 0:16]
                        a1 = a1 + rows_v[o + t, 16:32]
                    return a0, a1, cnt

                z = jnp.zeros((LANES,), jnp.float32)
                zi = jnp.zeros((LANES,), jnp.int32)
                a0, a1, cnt = lax.fori_loop(0, kmax, kbody, (z, z, zi))
                denom = jnp.maximum(jnp.sum(cnt).astype(jnp.float32), 1.0)
                inv = 1.0 / denom
                row_out = g * r_chunk + r
                out_v[row_out, 0:16] = a0 * inv
                out_v[row_out, 16:32] = a1 * inv

        nt = ng // 2
        fire(0, 0)

        def tbody(t, _):
            g0 = t * 2
            fire(g0 + 1, 1)
            drain(0)
            reduce(g0, 0)

            @pl.when(t < nt - 1)
            def _():
                fire(g0 + 2, 0)

            drain(1)
            reduce(g0 + 1, 1)
            return 0

        lax.fori_loop(0, nt, tbody, 0)
        pltpu.sync_copy(out_v, out_hbm.at[pl.ds(wid * rw, rw)])

    return k(x_flat, table)


def kernel(x, table):
    batch, hist = x.shape
    _, dim = table.shape
    hist_pad = ((hist + LANES - 1) // LANES) * LANES
    x_pad = jnp.pad(x.astype(jnp.int32), ((0, 0), (0, hist_pad - hist)))
    return _mean_emb(
        x_pad.reshape(-1), table, batch=batch, hist_pad=hist_pad, dim=dim
    )


# speedup vs baseline: 1.5418x; 1.5418x over previous
import functools

import jax
import jax.numpy as jnp
from jax import lax
from jax.experimental import pallas as pl
from jax.experimental.pallas import tpu as pltpu
from jax.experimental.pallas import tpu_sc as plsc

NC = 2
NS = 16
NW = NC * NS
LANES = 16


@functools.partial(jax.jit, static_argnames=("batch", "hist_pad", "dim"))
def _mean_emb(x_flat, table, *, batch, hist_pad, dim):
    rw = batch // NW
    r_chunk = 4
    ng = rw // r_chunk
    ch = 104
    nch = r_chunk * hist_pad // ch
    kmax = hist_pad // LANES

    mesh = plsc.VectorSubcoreMesh(core_axis_name="c", subcore_axis_name="s")

    @functools.partial(
        pl.kernel,
        out_type=jax.ShapeDtypeStruct((batch, dim), jnp.float32),
        mesh=mesh,
        compiler_params=pltpu.CompilerParams(
            needs_layout_passes=False, use_tc_tiling_on_sc=False
        ),
        scratch_types=[
            pltpu.VMEM((rw * hist_pad,), jnp.int32),
            pltpu.VMEM((r_chunk * hist_pad, dim), jnp.float32),
            pltpu.VMEM((r_chunk * hist_pad, dim), jnp.float32),
            pltpu.VMEM((rw, dim), jnp.float32),
            pltpu.SemaphoreType.DMA,
            pltpu.SemaphoreType.DMA,
        ],
    )
    def k(x_hbm, table_hbm, out_hbm, idx_v, rows0, rows1, out_v, sem0, sem1):
        wid = lax.axis_index("s") * NC + lax.axis_index("c")
        ibase = wid * (rw * hist_pad)
        pltpu.sync_copy(x_hbm.at[pl.ds(ibase, rw * hist_pad)], idx_v)

        rows_bufs = (rows0, rows1)
        sems = (sem0, sem1)

        def fire(g, b):
            goff = g * (r_chunk * hist_pad)
            for j in range(nch):
                pltpu.async_copy(
                    table_hbm.at[idx_v.at[pl.ds(goff + j * ch, ch)]],
                    rows_bufs[b].at[pl.ds(j * ch, ch)],
                    sems[b],
                )

        def drain(b):
            for j in range(nch):
                pltpu.make_async_copy(
                    table_hbm.at[idx_v.at[pl.ds(j * ch, ch)]],
                    rows_bufs[b].at[pl.ds(j * ch, ch)],
                    sems[b],
                ).wait()

        def reduce(g, b):
            rows_v = rows_bufs[b]
            goff = g * (r_chunk * hist_pad)
            for r in range(r_chunk):
                rbase = r * hist_pad

                def kbody(kk, carry, rbase=rbase):
                    a0, a1, cnt = carry
                    o = rbase + kk * LANES
                    idx = idx_v[pl.ds(goff + o, LANES)]
                    cnt = cnt + plsc.all_reduce_population_count(idx != 0)
                    for t in range(LANES):
                        a0 = a0 + rows_v[o + t, 0:16]
                        a1 = a1 + rows_v[o + t, 16:32]
                    return a0, a1, cnt

                z = jnp.zeros((LANES,), jnp.float32)
                zi = jnp.zeros((LANES,), jnp.int32)
                a0, a1, cnt = lax.fori_loop(0, kmax, kbody, (z, z, zi))
                inv = 1.0 / jnp.maximum(cnt.astype(jnp.float32), 1.0)
                row_out = g * r_chunk + r
                out_v[row_out, 0:16] = a0 * inv
                out_v[row_out, 16:32] = a1 * inv

        nt = ng // 2
        fire(0, 0)

        def tbody(t, _):
            g0 = t * 2
            fire(g0 + 1, 1)
            drain(0)
            reduce(g0, 0)

            @pl.when(t < nt - 1)
            def _():
                fire(g0 + 2, 0)

            drain(1)
            reduce(g0 + 1, 1)
            return 0

        lax.fori_loop(0, nt, tbody, 0)
        pltpu.sync_copy(out_v, out_hbm.at[pl.ds(wid * rw, rw)])

    return k(x_flat, table)


def kernel(x, table):
    batch, hist = x.shape
    _, dim = table.shape
    hist_pad = ((hist + LANES - 1) // LANES) * LANES
    x_pad = jnp.pad(x.astype(jnp.int32), ((0, 0), (0, hist_pad - hist)))
    return _mean_emb(
        x_pad.reshape(-1), table, batch=batch, hist_pad=hist_pad, dim=dim
    )


# speedup vs baseline: 2.4405x; 1.5829x over previous
import functools

import jax
import jax.numpy as jnp
from jax import lax
from jax.experimental import pallas as pl
from jax.experimental.pallas import tpu as pltpu
from jax.experimental.pallas import tpu_sc as plsc

NC = 2
NS = 16
NW = NC * NS
LANES = 16


def _tree_sum(vs):
    while len(vs) > 1:
        vs = [vs[i] + vs[i + 1] for i in range(0, len(vs) - 1, 2)] + (
            [vs[-1]] if len(vs) % 2 else []
        )
    return vs[0]


@functools.partial(jax.jit, static_argnames=("batch", "hist", "dim"))
def _mean_emb(x_flat, table, *, batch, hist, dim):
    rw = batch // NW
    r_chunk = 4
    ng = rw // r_chunk
    ch = 80
    nch = r_chunk * hist // ch
    kfull = hist // LANES
    tail = hist - kfull * LANES

    mesh = plsc.VectorSubcoreMesh(core_axis_name="c", subcore_axis_name="s")

    @functools.partial(
        pl.kernel,
        out_type=jax.ShapeDtypeStruct((batch, dim), jnp.float32),
        mesh=mesh,
        compiler_params=pltpu.CompilerParams(
            needs_layout_passes=False, use_tc_tiling_on_sc=False
        ),
        scratch_types=[
            pltpu.VMEM((rw * hist + (LANES - tail if tail else 0),), jnp.int32),
            pltpu.VMEM((r_chunk * hist, dim), jnp.float32),
            pltpu.VMEM((r_chunk * hist, dim), jnp.float32),
            pltpu.VMEM((rw, dim), jnp.float32),
            pltpu.SemaphoreType.DMA,
            pltpu.SemaphoreType.DMA,
        ],
    )
    def k(x_hbm, table_hbm, out_hbm, idx_v, rows0, rows1, out_v, sem0, sem1):
        wid = lax.axis_index("s") * NC + lax.axis_index("c")
        ibase = wid * (rw * hist)
        pltpu.sync_copy(x_hbm.at[pl.ds(ibase, rw * hist)], idx_v.at[pl.ds(0, rw * hist)])

        rows_bufs = (rows0, rows1)
        sems = (sem0, sem1)

        def fire(g, b):
            goff = g * (r_chunk * hist)
            for j in range(nch):
                pltpu.async_copy(
                    table_hbm.at[idx_v.at[pl.ds(goff + j * ch, ch)]],
                    rows_bufs[b].at[pl.ds(j * ch, ch)],
                    sems[b],
                )

        def drain(b):
            for j in range(nch):
                pltpu.make_async_copy(
                    table_hbm.at[idx_v.at[pl.ds(j * ch, ch)]],
                    rows_bufs[b].at[pl.ds(j * ch, ch)],
                    sems[b],
                ).wait()

        lane_lt_tail = lax.iota(jnp.int32, LANES) < tail

        def reduce(g, b):
            rows_v = rows_bufs[b]
            goff = g * (r_chunk * hist)
            for r in range(r_chunk):
                rbase = r * hist

                def kbody(kk, carry, rbase=rbase):
                    a0, a1, cnt = carry
                    o = rbase + kk * LANES
                    idx = idx_v[pl.ds(goff + o, LANES)]
                    cnt = cnt + plsc.all_reduce_population_count(idx != 0)
                    a0 = a0 + _tree_sum(
                        [rows_v[o + t, 0:16] for t in range(LANES)]
                    )
                    a1 = a1 + _tree_sum(
                        [rows_v[o + t, 16:32] for t in range(LANES)]
                    )
                    return a0, a1, cnt

                z = jnp.zeros((LANES,), jnp.float32)
                zi = jnp.zeros((LANES,), jnp.int32)
                a0, a1, cnt = lax.fori_loop(0, kfull, kbody, (z, z, zi))
                if tail:
                    o = rbase + kfull * LANES
                    idx = idx_v[pl.ds(goff + o, LANES)]
                    cnt = cnt + plsc.all_reduce_population_count(
                        (idx != 0) & lane_lt_tail
                    )
                    a0 = a0 + _tree_sum(
                        [rows_v[o + t, 0:16] for t in range(tail)]
                    )
                    a1 = a1 + _tree_sum(
                        [rows_v[o + t, 16:32] for t in range(tail)]
                    )
                inv = 1.0 / jnp.maximum(cnt.astype(jnp.float32), 1.0)
                row_out = g * r_chunk + r
                out_v[row_out, 0:16] = a0 * inv
                out_v[row_out, 16:32] = a1 * inv

        nt = ng // 2
        fire(0, 0)

        def tbody(t, _):
            g0 = t * 2
            fire(g0 + 1, 1)
            drain(0)
            reduce(g0, 0)

            @pl.when(t < nt - 1)
            def _():
                fire(g0 + 2, 0)

            drain(1)
            reduce(g0 + 1, 1)
            return 0

        lax.fori_loop(0, nt, tbody, 0)
        pltpu.sync_copy(out_v, out_hbm.at[pl.ds(wid * rw, rw)])

    return k(x_flat, table)


def kernel(x, table):
    batch, hist = x.shape
    _, dim = table.shape
    return _mean_emb(
        x.astype(jnp.int32).reshape(-1), table, batch=batch, hist=hist, dim=dim
    )
